# baseline (speedup 1.0000x reference)
import jax
import jax.numpy as jnp
from jax.experimental import pallas as pl


def _add_kernel(x_ref, w_ref, o_ref):
    o_ref[...] = x_ref[...] + w_ref[...]


def kernel(x, weight):
    batch, seq_len, dim = x.shape
    blk = 256
    grid = (seq_len // blk,)
    return pl.pallas_call(
        _add_kernel,
        grid=grid,
        in_specs=[
            pl.BlockSpec((batch, blk, dim), lambda s: (0, s, 0)),
            pl.BlockSpec((None, blk, dim), lambda s: (0, s, 0)),
        ],
        out_specs=pl.BlockSpec((batch, blk, dim), lambda s: (0, s, 0)),
        out_shape=jax.ShapeDtypeStruct(x.shape, x.dtype),
    )(x, weight[None, :seq_len, :])


# speedup vs baseline: 1.0046x; 1.0046x over previous
import jax
import jax.numpy as jnp
from jax.experimental import pallas as pl


def _add_kernel(x_ref, w_ref, o_ref):
    o_ref[...] = x_ref[...] + w_ref[...]


def kernel(x, weight):
    batch, seq_len, dim = x.shape
    blk = 512
    grid = (seq_len // blk,)
    return pl.pallas_call(
        _add_kernel,
        grid=grid,
        in_specs=[
            pl.BlockSpec((batch, blk, dim), lambda s: (0, s, 0)),
            pl.BlockSpec((None, blk, dim), lambda s: (0, s, 0)),
        ],
        out_specs=pl.BlockSpec((batch, blk, dim), lambda s: (0, s, 0)),
        out_shape=jax.ShapeDtypeStruct(x.shape, x.dtype),
    )(x, weight[None, :seq_len, :])
